# Initial kernel scaffold; baseline (speedup 1.0000x reference)
import functools

import jax
import jax.numpy as jnp
from jax import lax
from jax.experimental import pallas as pl
from jax.experimental.pallas import tpu as pltpu
from jax.experimental.pallas import tpu_sc as plsc

B = 16384
L = 50
EMB = 32
C = 3

NC = 2
NS = 16
NW = NC * NS
RPW = B // NW
NB = 8
NBLK = RPW // NB
IDX_PER_BLK = NB * L
CH = 80
NCH = IDX_PER_BLK // CH


def _sc_partial(xb, table, wt):
  mesh = plsc.VectorSubcoreMesh(core_axis_name="c", subcore_axis_name="s")

  @functools.partial(
      pl.kernel,
      mesh=mesh,
      out_type=jax.ShapeDtypeStruct((B, C, 16), jnp.float32),
      scratch_types=[
          pltpu.VMEM((NCH, CH), jnp.int32),
          pltpu.VMEM((IDX_PER_BLK, EMB), jnp.float32),
          pltpu.VMEM((L, C, EMB), jnp.float32),
          pltpu.VMEM((NB, C, 16), jnp.float32),
          pltpu.SemaphoreType.DMA,
      ],
  )
  def k(xb_hbm, table_hbm, wt_hbm, p_hbm, idx_v, rows_v, wt_v, out_v, sem):
    wid = lax.axis_index("s") * NC + lax.axis_index("c")
    pltpu.sync_copy(wt_hbm, wt_v)

    def block_body(kblk, _):
      pltpu.sync_copy(xb_hbm.at[wid, kblk], idx_v)
      handles = []
      for j in range(NCH):
        handles.append(
            pltpu.async_copy(
                table_hbm.at[idx_v.at[j]],
                rows_v.at[pl.ds(j * CH, CH)],
                sem,
            ))
      for h in handles:
        h.wait()

      acc = [[jnp.zeros((16,), jnp.float32) for _ in range(NB)]
             for _ in range(C)]
      for l in range(L):
        w = [(wt_v[l, c, 0:16], wt_v[l, c, 16:32]) for c in range(C)]
        for j in range(NB):
          r0 = rows_v[l * NB + j, 0:16]
          r1 = rows_v[l * NB + j, 16:32]
          for c in range(C):
            acc[c][j] = acc[c][j] + r0 * w[c][0] + r1 * w[c][1]

      for j in range(NB):
        for c in range(C):
          out_v[j, c, :] = acc[c][j]
      pltpu.sync_copy(out_v, p_hbm.at[pl.ds(wid * RPW + kblk * NB, NB)])
      return ()

    lax.fori_loop(0, NBLK, block_body, ())

  return k(xb, table, wt)


def _tc_finish(p, bias):
  def body(p_ref, b_ref, o_ref):
    z = jnp.sum(p_ref[...], axis=-1) + b_ref[...]
    m = jnp.max(z, axis=-1, keepdims=True)
    e = jnp.exp(z - m)
    o_ref[...] = (z - m) - jnp.log(jnp.sum(e, axis=-1, keepdims=True))

  return pl.pallas_call(
      body,
      out_shape=jax.ShapeDtypeStruct((B, C), jnp.float32),
  )(p, bias.reshape(1, C))


def kernel(x, table, W, b):
  xb = (
      x.astype(jnp.int32)
      .reshape(NW, NBLK, NB, L)
      .transpose(0, 1, 3, 2)
      .reshape(NW, NBLK, NCH, CH)
  )
  wt = W.reshape(C, L, EMB).transpose(1, 0, 2)
  p = _sc_partial(xb, table, wt)
  return _tc_finish(p, b)


# speedup vs baseline: 23.4620x; 23.4620x over previous
import functools

import jax
import jax.numpy as jnp
from jax import lax
from jax.experimental import pallas as pl
from jax.experimental.pallas import tpu as pltpu
from jax.experimental.pallas import tpu_sc as plsc

B = 16384
L = 50
EMB = 32
C = 3

NC = 2
NS = 16
NW = NC * NS
RPW = B // NW
NB = 8
NBLK = RPW // NB
IDX_PER_BLK = NB * L
CH = 80
NCH = IDX_PER_BLK // CH


def _sc_partial(xb, table, wt):
  mesh = plsc.VectorSubcoreMesh(core_axis_name="c", subcore_axis_name="s")

  @functools.partial(
      pl.kernel,
      mesh=mesh,
      compiler_params=pltpu.CompilerParams(use_tc_tiling_on_sc=False),
      out_type=jax.ShapeDtypeStruct((B, C, 16), jnp.float32),
      scratch_types=[
          pltpu.VMEM((NCH, CH), jnp.int32),
          pltpu.VMEM((IDX_PER_BLK, EMB), jnp.float32),
          pltpu.VMEM((L, C, EMB), jnp.float32),
          pltpu.VMEM((NB, C, 16), jnp.float32),
          pltpu.SemaphoreType.DMA,
      ],
  )
  def k(xb_hbm, table_hbm, wt_hbm, p_hbm, idx_v, rows_v, wt_v, out_v, sem):
    wid = lax.axis_index("s") * NC + lax.axis_index("c")
    pltpu.sync_copy(wt_hbm, wt_v)

    def block_body(kblk, _):
      pltpu.sync_copy(xb_hbm.at[wid, kblk], idx_v)
      handles = []
      for j in range(NCH):
        handles.append(
            pltpu.async_copy(
                table_hbm.at[idx_v.at[j]],
                rows_v.at[pl.ds(j * CH, CH)],
                sem,
            ))
      for h in handles:
        h.wait()

      acc = [[jnp.zeros((16,), jnp.float32) for _ in range(NB)]
             for _ in range(C)]
      for l in range(L):
        w = [(wt_v[l, c, 0:16], wt_v[l, c, 16:32]) for c in range(C)]
        for j in range(NB):
          r0 = rows_v[l * NB + j, 0:16]
          r1 = rows_v[l * NB + j, 16:32]
          for c in range(C):
            acc[c][j] = acc[c][j] + r0 * w[c][0] + r1 * w[c][1]

      for j in range(NB):
        for c in range(C):
          out_v[j, c, :] = acc[c][j]
      pltpu.sync_copy(out_v, p_hbm.at[pl.ds(wid * RPW + kblk * NB, NB)])
      return ()

    lax.fori_loop(0, NBLK, block_body, ())

  return k(xb, table, wt)


def _tc_finish(p, bias):
  blk = 1024

  def body(p_ref, b_ref, o_ref):
    z = jnp.sum(p_ref[...], axis=-1) + b_ref[...]
    m = jnp.max(z, axis=-1, keepdims=True)
    e = jnp.exp(z - m)
    o_ref[...] = (z - m) - jnp.log(jnp.sum(e, axis=-1, keepdims=True))

  return pl.pallas_call(
      body,
      grid=(B // blk,),
      in_specs=[
          pl.BlockSpec((blk, C, 16), lambda i: (i, 0, 0)),
          pl.BlockSpec((1, C), lambda i: (0, 0)),
      ],
      out_specs=pl.BlockSpec((blk, C), lambda i: (i, 0)),
      out_shape=jax.ShapeDtypeStruct((B, C), jnp.float32),
  )(p, bias.reshape(1, C))


def kernel(x, table, W, b):
  xb = (
      x.astype(jnp.int32)
      .reshape(NW, NBLK, NB, L)
      .transpose(0, 1, 3, 2)
      .reshape(NW, NBLK, NCH, CH)
  )
  wt = W.reshape(C, L, EMB).transpose(1, 0, 2)
  p = _sc_partial(xb, table, wt)
  return _tc_finish(p, b)


# speedup vs baseline: 26.7317x; 1.1394x over previous
import functools

import jax
import jax.numpy as jnp
from jax import lax
from jax.experimental import pallas as pl
from jax.experimental.pallas import tpu as pltpu
from jax.experimental.pallas import tpu_sc as plsc

B = 16384
L = 50
EMB = 32
C = 3

NC = 2
NS = 16
NW = NC * NS
RPW = B // NW
NB = 8
NBLK = RPW // NB
FPB = NB * L
LU = 5


def _sc_partial(x, table, wt):
  mesh = plsc.VectorSubcoreMesh(core_axis_name="c", subcore_axis_name="s")

  @functools.partial(
      pl.kernel,
      mesh=mesh,
      compiler_params=pltpu.CompilerParams(use_tc_tiling_on_sc=False),
      out_type=jax.ShapeDtypeStruct((B, 128), jnp.float32),
      scratch_types=[
          pltpu.VMEM((NB, L), jnp.int32),
          pltpu.VMEM((NB, L), jnp.int32),
          pltpu.VMEM((FPB, EMB), jnp.float32),
          pltpu.VMEM((FPB, EMB), jnp.float32),
          pltpu.VMEM((L, C, EMB), jnp.float32),
          pltpu.VMEM((NB, 128), jnp.float32),
          pltpu.SemaphoreType.DMA,
          pltpu.SemaphoreType.DMA,
      ],
  )
  def k(x_hbm, table_hbm, wt_hbm, p_hbm, idx_a, idx_b, rows_a, rows_b,
        wt_v, out_v, sem_a, sem_b):
    wid = lax.axis_index("s") * NC + lax.axis_index("c")
    row0 = wid * RPW
    pltpu.sync_copy(wt_hbm, wt_v)

    zero = jnp.zeros((16,), jnp.float32)
    for j in range(NB):
      for h in range(C, 8):
        out_v[j, pl.ds(h * 16, 16)] = zero

    def fire(kblk, idx, rows, sem):
      pltpu.sync_copy(x_hbm.at[pl.ds(row0 + kblk * NB, NB)], idx)
      for j in range(NB):
        pltpu.async_copy(
            table_hbm.at[idx.at[j]], rows.at[pl.ds(j * L, L)], sem)

    def drain(rows, sem):
      pltpu.make_async_copy(table_hbm.at[pl.ds(0, FPB)], rows, sem).wait()

    def compute(kblk, rows):
      def lbody(li, acc):
        accs = list(acc)
        for u in range(LU):
          lpos = li * LU + u
          w = [(wt_v[lpos, c, 0:16], wt_v[lpos, c, 16:32]) for c in range(C)]
          for j in range(NB):
            f = j * L + lpos
            r0 = rows[f, 0:16]
            r1 = rows[f, 16:32]
            for c in range(C):
              accs[c * NB + j] = accs[c * NB + j] + r0 * w[c][0] + r1 * w[c][1]
        return tuple(accs)

      acc = lax.fori_loop(0, L // LU, lbody, (zero,) * (C * NB))
      for j in range(NB):
        for c in range(C):
          out_v[j, pl.ds(c * 16, 16)] = acc[c * NB + j]
      pltpu.sync_copy(out_v, p_hbm.at[pl.ds(row0 + kblk * NB, NB)])

    fire(0, idx_a, rows_a, sem_a)

    def pair_body(i, _):
      k2 = i * 2
      drain(rows_a, sem_a)
      fire(k2 + 1, idx_b, rows_b, sem_b)
      compute(k2, rows_a)
      drain(rows_b, sem_b)

      @pl.when(k2 + 2 < NBLK)
      def _():
        fire(k2 + 2, idx_a, rows_a, sem_a)

      compute(k2 + 1, rows_b)
      return ()

    lax.fori_loop(0, NBLK // 2, pair_body, ())

  return k(x, table, wt)


def _tc_finish(p, bias):
  blk = 2048

  def body(p_ref, b_ref, o_ref):
    z = [
        jnp.sum(p_ref[:, c * 16:(c + 1) * 16], axis=-1, keepdims=True)
        + b_ref[0, c]
        for c in range(C)
    ]
    m = jnp.maximum(jnp.maximum(z[0], z[1]), z[2])
    s = jnp.exp(z[0] - m) + jnp.exp(z[1] - m) + jnp.exp(z[2] - m)
    ln = jnp.log(s)
    for c in range(C):
      o_ref[:, c:c + 1] = z[c] - m - ln

  return pl.pallas_call(
      body,
      grid=(B // blk,),
      in_specs=[
          pl.BlockSpec((blk, 128), lambda i: (i, 0)),
          pl.BlockSpec((1, C), lambda i: (0, 0)),
      ],
      out_specs=pl.BlockSpec((blk, C), lambda i: (i, 0)),
      out_shape=jax.ShapeDtypeStruct((B, C), jnp.float32),
  )(p, bias.reshape(1, C))


def kernel(x, table, W, b):
  wt = W.reshape(C, L, EMB).transpose(1, 0, 2)
  p = _sc_partial(x.astype(jnp.int32), table, wt)
  return _tc_finish(p, b)
